# baseline (speedup 1.0000x reference)
import functools

import jax
import jax.numpy as jnp
from jax import lax
from jax.experimental import pallas as pl
from jax.experimental.pallas import tpu as pltpu
from jax.experimental.pallas import tpu_sc as plsc

ALPHA = 0.025
NC = 2
NS = 16
NW = NC * NS
LANES = 16

N = 1000000
D = 64
B = 16384
K = 5
T = K + 1

BW = 128
NBLK = N // BW
LAST_BASE = NBLK * BW
NBI = NBLK // NW + 1
CAP = 3072

_mesh = plsc.VectorSubcoreMesh(core_axis_name="c", subcore_axis_name="s")
_sc_params = pltpu.CompilerParams(
    use_tc_tiling_on_sc=True, needs_layout_passes=False)


def _wid():
    return lax.axis_index("s") * NC + lax.axis_index("c")


def _ext(vec, l, lane):
    return jnp.sum(jnp.where(lane == l, vec, jnp.zeros_like(vec)))


def _ext_at(ref, i, lane):
    base = (i // LANES) * LANES
    vec = ref[pl.ds(base, LANES)]
    return jnp.sum(jnp.where(lane == i % LANES, vec, 0))


@functools.partial(
    pl.kernel,
    out_type=(
        jax.ShapeDtypeStruct((B, D), jnp.float32),
        jax.ShapeDtypeStruct((T * B, D), jnp.float32),
    ),
    mesh=_mesh,
    compiler_params=_sc_params,
    scratch_types=[
        pltpu.VMEM((T * B + LANES,), jnp.int32),
        pltpu.VMEM((CAP + LANES,), jnp.int32),
        pltpu.VMEM((CAP + LANES,), jnp.int32),
        pltpu.VMEM((CAP + LANES,), jnp.int32),
        pltpu.VMEM((CAP + LANES,), jnp.int32),
        pltpu.VMEM((64, BW), jnp.float32),
        pltpu.VMEM((LANES, D), jnp.float32),
        pltpu.SemaphoreType.DMA,
        pltpu.SemaphoreType.DMA,
    ],
)
def _k1_sweep(evt, ect, u, v, negf, vecu_out, vecv_out,
              stage, my_tgt, my_slot, bl_col, bl_slot, blockbuf, ring,
              sem_b, sem_w):
    wid = _wid()
    lane = lax.iota(jnp.int32, LANES)

    def run_phase(tbl, nstage, out_ref, is_ctx):
        def cbody(i, cnt):
            tv = stage[pl.ds(i * LANES, LANES)]
            msk = ((tv >> 7) & (NW - 1)) == wid
            return cnt + jnp.sum(msk.astype(jnp.int32))
        cnt = lax.fori_loop(0, nstage // LANES, cbody, 0)
        nrounds = (cnt + CAP - 1) // CAP

        def round_body(r, carry0):
            lo = r * CAP
            hi = lo + CAP

            def sbody(i, carry):
                gc, oc = carry
                tv = stage[pl.ds(i * LANES, LANES)]
                msk = ((tv >> 7) & (NW - 1)) == wid
                pref = plsc.cumsum(msk.astype(jnp.int32))
                idx = gc + pref - 1
                keep = msk & (idx >= lo) & (idx < hi)
                plsc.store_compressed(my_tgt.at[pl.ds(oc, LANES)], tv,
                                      mask=keep)
                plsc.store_compressed(my_slot.at[pl.ds(oc, LANES)],
                                      lane + i * LANES, mask=keep)
                return (gc + jnp.sum(msk.astype(jnp.int32)),
                        oc + jnp.sum(keep.astype(jnp.int32)))

            _, rc = lax.fori_loop(0, nstage // LANES, sbody, (0, 0))

            def block_body(bi, carry):
                blk = bi * NW + wid

                @pl.when(blk < NBLK)
                def _():
                    def mbody(j, mb):
                        mt = my_tgt[pl.ds(j * LANES, LANES)]
                        valid = lane < (rc - j * LANES)
                        bm = ((mt >> 7) == blk) & valid
                        ms = my_slot[pl.ds(j * LANES, LANES)]
                        plsc.store_compressed(
                            bl_col.at[pl.ds(mb, LANES)], mt & (BW - 1),
                            mask=bm)
                        plsc.store_compressed(
                            bl_slot.at[pl.ds(mb, LANES)], ms, mask=bm)
                        return mb + jnp.sum(bm.astype(jnp.int32))
                    mb = lax.fori_loop(0, (rc + LANES - 1) // LANES, mbody, 0)

                    @pl.when(mb > 0)
                    def _():
                        pltpu.async_copy(
                            tbl.at[:, pl.ds(blk * BW, BW)], blockbuf,
                            sem_b).wait()

                        def ebody(i, carry2):
                            c = _ext_at(bl_col, i, lane)
                            s = _ext_at(bl_slot, i, lane)
                            li = i % LANES
                            @pl.when(i >= LANES)
                            def _():
                                pltpu.make_async_copy(
                                    ring.at[pl.ds(0, 1), :],
                                    out_ref.at[pl.ds(0, 1), :],
                                    sem_w).wait()
                            csp = jnp.full((LANES,), c, jnp.int32)
                            for q in range(D // LANES):
                                dvec = lane + q * LANES
                                vals = plsc.load_gather(blockbuf, [dvec, csp])
                                plsc.store_scatter(
                                    ring, [jnp.full((LANES,), li, jnp.int32),
                                           dvec], vals)
                            if is_ctx:
                                kk = s - B
                                dstrow = jnp.where(
                                    s < B, s,
                                    (1 + kk % K) * B + kk // K)
                            else:
                                dstrow = s
                            pltpu.async_copy(
                                ring.at[pl.ds(li, 1), :],
                                out_ref.at[pl.ds(dstrow, 1), :], sem_w)
                            return carry2

                        lax.fori_loop(0, mb, ebody, 0)
                        ndrain = jnp.minimum(mb, LANES)

                        def dbody(i, carry3):
                            pltpu.make_async_copy(
                                ring.at[pl.ds(0, 1), :],
                                out_ref.at[pl.ds(0, 1), :], sem_w).wait()
                            return carry3
                        lax.fori_loop(0, ndrain, dbody, 0)
                return carry

            lax.fori_loop(0, NBI, block_body, 0)
            return carry0

        lax.fori_loop(0, nrounds, round_body, 0)

    pltpu.sync_copy(u, stage.at[pl.ds(0, B)])
    run_phase(evt, B, vecu_out, False)
    pltpu.sync_copy(v, stage.at[pl.ds(0, B)])
    pltpu.sync_copy(negf, stage.at[pl.ds(B, K * B)])
    run_phase(ect, T * B, vecv_out, True)


BLK = 2048


def _k2_body(vecu_ref, *refs):
    vv_refs = refs[:T]
    u_ref, v_ref, neg_ref, evtail_ref, ectail_ref = refs[T:T + 5]
    verr_ref, tail_ref = refs[T + 5:]
    i = pl.program_id(0)

    rmask = (lax.broadcasted_iota(jnp.int32, (BW, D), 0)
             < (N - LAST_BASE)).astype(jnp.float32)
    evtail = jnp.transpose(evtail_ref[...]) * rmask
    ectail = jnp.transpose(ectail_ref[...]) * rmask
    u_blk = u_ref[...]
    negs = neg_ref[...]
    tgt_cols = [v_ref[...]] + [
        jnp.transpose(negs[t - 1:t, :]) for t in range(1, T)]

    def onehot(idx2):
        c = lax.broadcasted_iota(jnp.int32, (BLK, BW), 1) + LAST_BASE
        return jnp.where(idx2 == c, 1.0, 0.0)

    mu = onehot(u_blk)
    vu = vecu_ref[...]
    vu = jnp.where(u_blk >= LAST_BASE,
                   jnp.dot(mu, evtail, preferred_element_type=jnp.float32),
                   vu)
    acc = jnp.zeros((BLK, D), jnp.float32)
    for t in range(T):
        vvt = vv_refs[t][...]
        tc = tgt_cols[t]
        vvt = jnp.where(
            tc >= LAST_BASE,
            jnp.dot(onehot(tc), ectail,
                    preferred_element_type=jnp.float32),
            vvt)
        dot = jnp.sum(vu * vvt, axis=-1, keepdims=True)
        f = 1.0 / (1.0 + jnp.exp(-dot))
        lab = 1.0 if t == 0 else 0.0
        g = ALPHA * (lab - f)
        acc = acc + g * vvt
    verr_ref[...] = acc
    contrib = jnp.dot(mu.T, acc, preferred_element_type=jnp.float32)

    @pl.when(i == 0)
    def _():
        tail_ref[...] = evtail + contrib

    @pl.when(i > 0)
    def _():
        tail_ref[...] = tail_ref[...] + contrib


def _k2_dense(vecu, vecvf, u, v, negT, evt, ect):
    nsteps = B // BLK
    in_specs = [pl.BlockSpec((BLK, D), lambda i: (i, 0))]
    for t in range(T):
        in_specs.append(
            pl.BlockSpec((BLK, D), lambda i, t=t: (t * nsteps + i, 0)))
    in_specs += [
        pl.BlockSpec((BLK, 1), lambda i: (i, 0)),
        pl.BlockSpec((BLK, 1), lambda i: (i, 0)),
        pl.BlockSpec((K, BLK), lambda i: (0, i)),
        pl.BlockSpec((D, BW), lambda i: (0, NBLK)),
        pl.BlockSpec((D, BW), lambda i: (0, NBLK)),
    ]
    return pl.pallas_call(
        _k2_body,
        grid=(nsteps,),
        in_specs=in_specs,
        out_specs=[
            pl.BlockSpec((BLK, D), lambda i: (i, 0)),
            pl.BlockSpec((BW, D), lambda i: (0, 0)),
        ],
        out_shape=[
            jax.ShapeDtypeStruct((B, D), jnp.float32),
            jax.ShapeDtypeStruct((BW, D), jnp.float32),
        ],
    )(vecu, *([vecvf] * T), u[:, None], v[:, None], negT, evt, ect)


@functools.partial(
    pl.kernel,
    out_type=jax.ShapeDtypeStruct((D, N), jnp.float32),
    mesh=_mesh,
    compiler_params=_sc_params,
    scratch_types=[
        pltpu.VMEM((B,), jnp.int32),
        pltpu.VMEM((B + LANES,), jnp.int32),
        pltpu.VMEM((B + LANES,), jnp.int32),
        pltpu.VMEM((B + LANES,), jnp.int32),
        pltpu.VMEM((B + LANES,), jnp.int32),
        pltpu.VMEM((64, BW), jnp.float32),
        pltpu.VMEM((64, BW), jnp.float32),
        pltpu.VMEM((LANES, D), jnp.float32),
        pltpu.SemaphoreType.DMA,
        pltpu.SemaphoreType.DMA,
        pltpu.SemaphoreType.DMA,
    ],
)
def _k3_fused(evt, u, verr, outt, u_all, my_eid, my_row, bl_col, bl_eid,
              bufa, bufb, vring, sem_i, sem_o, sem_v):
    wid = _wid()
    lane = lax.iota(jnp.int32, LANES)

    pltpu.sync_copy(u, u_all)

    def scan_body(i, off):
        uvec = u_all[pl.ds(i * LANES, LANES)]
        msk = ((uvec >> 7) & (NW - 1)) == wid
        plsc.store_compressed(my_eid.at[pl.ds(off, LANES)],
                              lane + i * LANES, mask=msk)
        plsc.store_compressed(my_row.at[pl.ds(off, LANES)], uvec, mask=msk)
        return off + jnp.sum(msk.astype(jnp.int32))

    m = lax.fori_loop(0, B // LANES, scan_body, 0)

    def process(buf, blk):
        def mbody(j, mb):
            mt = my_row[pl.ds(j * LANES, LANES)]
            valid = lane < (m - j * LANES)
            bm = ((mt >> 7) == blk) & valid
            me = my_eid[pl.ds(j * LANES, LANES)]
            plsc.store_compressed(bl_col.at[pl.ds(mb, LANES)],
                                  mt & (BW - 1), mask=bm)
            plsc.store_compressed(bl_eid.at[pl.ds(mb, LANES)], me, mask=bm)
            return mb + jnp.sum(bm.astype(jnp.int32))
        mb = lax.fori_loop(0, (m + LANES - 1) // LANES, mbody, 0)

        @pl.when(mb > 0)
        def _():
            def fire(i, carry):
                e = _ext_at(bl_eid, i, lane)
                pltpu.async_copy(verr.at[pl.ds(e, 1), :],
                                 vring.at[pl.ds(i % LANES, 1), :], sem_v)
                return carry
            lax.fori_loop(0, jnp.minimum(mb, LANES), fire, 0)

            def ebody(i, carry2):
                pltpu.make_async_copy(
                    verr.at[pl.ds(0, 1), :], vring.at[pl.ds(0, 1), :],
                    sem_v).wait()
                c = _ext_at(bl_col, i, lane)
                li = i % LANES
                csp = jnp.full((LANES,), c, jnp.int32)
                for q in range(D // LANES):
                    dvec = lane + q * LANES
                    vals = vring[li, pl.ds(q * LANES, LANES)]
                    plsc.addupdate_scatter(buf, [dvec, csp], vals)
                @pl.when(i + LANES < mb)
                def _():
                    e2 = _ext_at(bl_eid, i + LANES, lane)
                    pltpu.async_copy(verr.at[pl.ds(e2, 1), :],
                                     vring.at[pl.ds(li, 1), :], sem_v)
                return carry2
            lax.fori_loop(0, mb, ebody, 0)

    nfull = NBLK // NW + jnp.where(wid < NBLK % NW, 1, 0)

    @pl.when(nfull > 0)
    def _():
        pltpu.async_copy(evt.at[:, pl.ds(wid * BW, BW)], bufa, sem_i)

        def bbody(bi, carry):
            blk = bi * NW + wid
            nxt = blk + NW

            def go(cur, oth):
                pltpu.make_async_copy(
                    evt.at[:, pl.ds(0, BW)], cur, sem_i).wait()
                @pl.when(bi >= 1)
                def _():
                    pltpu.make_async_copy(
                        cur, outt.at[:, pl.ds(0, BW)], sem_o).wait()
                @pl.when(nxt < NBLK)
                def _():
                    pltpu.async_copy(
                        evt.at[:, pl.ds(nxt * BW, BW)], oth, sem_i)
                process(cur, blk)
                pltpu.async_copy(cur, outt.at[:, pl.ds(blk * BW, BW)], sem_o)

            @pl.when(bi % 2 == 0)
            def _():
                go(bufa, bufb)

            @pl.when(bi % 2 == 1)
            def _():
                go(bufb, bufa)
            return carry

        lax.fori_loop(0, nfull, bbody, 0)

        pltpu.make_async_copy(
            bufa, outt.at[:, pl.ds(0, BW)], sem_o).wait()


def kernel(emb_vertex, emb_context, u, v, neg):
    u = u.astype(jnp.int32)
    v = v.astype(jnp.int32)
    negf = neg.astype(jnp.int32).reshape(-1)
    evt = emb_vertex.T
    ect = emb_context.T
    vecu, vecvf = _k1_sweep(evt, ect, u, v, negf)
    verr, tail = _k2_dense(vecu, vecvf, u, v, neg.astype(jnp.int32).T,
                           evt, ect)
    outt = _k3_fused(evt, u, verr)
    out = outt.T
    return lax.dynamic_update_slice(out, tail[:N - LAST_BASE], (LAST_BASE, 0))


# speedup vs baseline: 1.2568x; 1.2568x over previous
import functools

import jax
import jax.numpy as jnp
from jax import lax
from jax.experimental import pallas as pl
from jax.experimental.pallas import tpu as pltpu
from jax.experimental.pallas import tpu_sc as plsc

ALPHA = 0.025
NC = 2
NS = 16
NW = NC * NS
LANES = 16

N = 1000000
D = 64
B = 16384
K = 5
T = K + 1

BW = 128
NBLK = N // BW
LAST_BASE = NBLK * BW
NBI = NBLK // NW + 1
CAP = 3072

_mesh = plsc.VectorSubcoreMesh(core_axis_name="c", subcore_axis_name="s")
_sc_params = pltpu.CompilerParams(
    use_tc_tiling_on_sc=True, needs_layout_passes=False)


def _wid():
    return lax.axis_index("s") * NC + lax.axis_index("c")


def _ext(vec, l, lane):
    return jnp.sum(jnp.where(lane == l, vec, jnp.zeros_like(vec)))


def _ext_at(ref, i, lane):
    base = (i // LANES) * LANES
    vec = ref[pl.ds(base, LANES)]
    return jnp.sum(jnp.where(lane == i % LANES, vec, 0))


SUB = 3 * B


@functools.partial(
    pl.kernel,
    out_type=(
        jax.ShapeDtypeStruct((B, D), jnp.float32),
        jax.ShapeDtypeStruct((T * B, D), jnp.float32),
    ),
    mesh=_mesh,
    compiler_params=_sc_params,
    scratch_types=[
        pltpu.VMEM((SUB + LANES,), jnp.int32),
        pltpu.VMEM((CAP + LANES,), jnp.int32),
        pltpu.VMEM((CAP + LANES,), jnp.int32),
        pltpu.VMEM((CAP + LANES,), jnp.int32),
        pltpu.VMEM((CAP + LANES,), jnp.int32),
        pltpu.VMEM((CAP + LANES,), jnp.int32),
        pltpu.VMEM((CAP + LANES,), jnp.int32),
        pltpu.VMEM((64, BW), jnp.float32),
        pltpu.VMEM((LANES, D), jnp.float32),
        pltpu.SemaphoreType.DMA,
        pltpu.SemaphoreType.DMA,
    ],
)
def _k1_sweep(evt, ect, u, v, negf, vecu_out, vecv_out,
              stage, my_tgt, my_slot, gr_tgt, gr_slot, bl_col, bl_slot,
              blockbuf, ring, sem_b, sem_w):
    wid = _wid()
    lane = lax.iota(jnp.int32, LANES)

    def run_phase(tbl, nstage, slot_base, out_ref, is_ctx):
        def cbody(i, cnt):
            tv = stage[pl.ds(i * LANES, LANES)]
            msk = ((tv >> 7) & (NW - 1)) == wid
            return cnt + jnp.sum(msk.astype(jnp.int32))
        cnt = lax.fori_loop(0, nstage // LANES, cbody, 0)
        nrounds = (cnt + CAP - 1) // CAP

        def round_body(r, carry0):
            lo = r * CAP
            hi = lo + CAP

            def sbody(i, carry):
                gc, oc = carry
                tv = stage[pl.ds(i * LANES, LANES)]
                msk = ((tv >> 7) & (NW - 1)) == wid
                pref = plsc.cumsum(msk.astype(jnp.int32))
                idx = gc + pref - 1
                keep = msk & (idx >= lo) & (idx < hi)
                plsc.store_compressed(my_tgt.at[pl.ds(oc, LANES)], tv,
                                      mask=keep)
                plsc.store_compressed(my_slot.at[pl.ds(oc, LANES)],
                                      lane + i * LANES + slot_base,
                                      mask=keep)
                return (gc + jnp.sum(msk.astype(jnp.int32)),
                        oc + jnp.sum(keep.astype(jnp.int32)))

            _, rc = lax.fori_loop(0, nstage // LANES, sbody, (0, 0))

            def group_body(g, carryg):
                def gbody(j, gb):
                    mt = my_tgt[pl.ds(j * LANES, LANES)]
                    valid = lane < (rc - j * LANES)
                    bi_of = ((mt >> 7) - wid) >> 5
                    gm = ((bi_of >> 4) == g) & valid
                    ms = my_slot[pl.ds(j * LANES, LANES)]
                    plsc.store_compressed(gr_tgt.at[pl.ds(gb, LANES)], mt,
                                          mask=gm)
                    plsc.store_compressed(gr_slot.at[pl.ds(gb, LANES)], ms,
                                          mask=gm)
                    return gb + jnp.sum(gm.astype(jnp.int32))
                gb = lax.fori_loop(0, (rc + LANES - 1) // LANES, gbody, 0)

                @pl.when(gb > 0)
                def _():
                    def block_body(bi2, carry):
                        blk = (g * LANES + bi2) * NW + wid

                        @pl.when(blk < NBLK)
                        def _():
                            def mbody(j, mb):
                                mt = gr_tgt[pl.ds(j * LANES, LANES)]
                                valid = lane < (gb - j * LANES)
                                bm = ((mt >> 7) == blk) & valid
                                ms = gr_slot[pl.ds(j * LANES, LANES)]
                                plsc.store_compressed(
                                    bl_col.at[pl.ds(mb, LANES)],
                                    mt & (BW - 1), mask=bm)
                                plsc.store_compressed(
                                    bl_slot.at[pl.ds(mb, LANES)], ms,
                                    mask=bm)
                                return mb + jnp.sum(bm.astype(jnp.int32))
                            mb = lax.fori_loop(
                                0, (gb + LANES - 1) // LANES, mbody, 0)

                            @pl.when(mb > 0)
                            def _():
                                pltpu.async_copy(
                                    tbl.at[:, pl.ds(blk * BW, BW)],
                                    blockbuf, sem_b).wait()

                                def ebody(i, carry2):
                                    c = _ext_at(bl_col, i, lane)
                                    s = _ext_at(bl_slot, i, lane)
                                    li = i % LANES
                                    @pl.when(i >= LANES)
                                    def _():
                                        pltpu.make_async_copy(
                                            ring.at[pl.ds(0, 1), :],
                                            out_ref.at[pl.ds(0, 1), :],
                                            sem_w).wait()
                                    csp = jnp.full((LANES,), c, jnp.int32)
                                    for q in range(D // LANES):
                                        dvec = lane + q * LANES
                                        vals = plsc.load_gather(
                                            blockbuf, [dvec, csp])
                                        plsc.store_scatter(
                                            ring,
                                            [jnp.full((LANES,), li,
                                                      jnp.int32), dvec],
                                            vals)
                                    if is_ctx:
                                        kk = s - B
                                        dstrow = jnp.where(
                                            s < B, s,
                                            (1 + kk % K) * B + kk // K)
                                    else:
                                        dstrow = s
                                    pltpu.async_copy(
                                        ring.at[pl.ds(li, 1), :],
                                        out_ref.at[pl.ds(dstrow, 1), :],
                                        sem_w)
                                    return carry2

                                lax.fori_loop(0, mb, ebody, 0)
                                ndrain = jnp.minimum(mb, LANES)

                                def dbody(i, carry3):
                                    pltpu.make_async_copy(
                                        ring.at[pl.ds(0, 1), :],
                                        out_ref.at[pl.ds(0, 1), :],
                                        sem_w).wait()
                                    return carry3
                                lax.fori_loop(0, ndrain, dbody, 0)
                        return carry

                    lax.fori_loop(0, LANES, block_body, 0)
                return carryg

            lax.fori_loop(0, LANES, group_body, 0)
            return carry0

        lax.fori_loop(0, nrounds, round_body, 0)

    pltpu.sync_copy(u, stage.at[pl.ds(0, B)])
    run_phase(evt, B, 0, vecu_out, False)
    pltpu.sync_copy(v, stage.at[pl.ds(0, B)])
    pltpu.sync_copy(negf.at[pl.ds(0, 2 * B)], stage.at[pl.ds(B, 2 * B)])
    run_phase(ect, SUB, 0, vecv_out, True)
    pltpu.sync_copy(negf.at[pl.ds(2 * B, 3 * B)], stage.at[pl.ds(0, 3 * B)])
    run_phase(ect, SUB, SUB, vecv_out, True)


BLK = 2048


def _k2_body(vecu_ref, *refs):
    vv_refs = refs[:T]
    u_ref, v_ref, neg_ref, evtail_ref, ectail_ref = refs[T:T + 5]
    verr_ref, tail_ref = refs[T + 5:]
    i = pl.program_id(0)

    rmask = (lax.broadcasted_iota(jnp.int32, (BW, D), 0)
             < (N - LAST_BASE)).astype(jnp.float32)
    evtail = jnp.transpose(evtail_ref[...]) * rmask
    ectail = jnp.transpose(ectail_ref[...]) * rmask
    u_blk = u_ref[...]
    negs = neg_ref[...]
    tgt_cols = [v_ref[...]] + [
        jnp.transpose(negs[t - 1:t, :]) for t in range(1, T)]

    def onehot(idx2):
        c = lax.broadcasted_iota(jnp.int32, (BLK, BW), 1) + LAST_BASE
        return jnp.where(idx2 == c, 1.0, 0.0)

    mu = onehot(u_blk)
    vu = vecu_ref[...]
    vu = jnp.where(u_blk >= LAST_BASE,
                   jnp.dot(mu, evtail, preferred_element_type=jnp.float32),
                   vu)
    acc = jnp.zeros((BLK, D), jnp.float32)
    for t in range(T):
        vvt = vv_refs[t][...]
        tc = tgt_cols[t]
        vvt = jnp.where(
            tc >= LAST_BASE,
            jnp.dot(onehot(tc), ectail,
                    preferred_element_type=jnp.float32),
            vvt)
        dot = jnp.sum(vu * vvt, axis=-1, keepdims=True)
        f = 1.0 / (1.0 + jnp.exp(-dot))
        lab = 1.0 if t == 0 else 0.0
        g = ALPHA * (lab - f)
        acc = acc + g * vvt
    verr_ref[...] = acc
    contrib = jnp.dot(mu.T, acc, preferred_element_type=jnp.float32)

    @pl.when(i == 0)
    def _():
        tail_ref[...] = evtail + contrib

    @pl.when(i > 0)
    def _():
        tail_ref[...] = tail_ref[...] + contrib


def _k2_dense(vecu, vecvf, u, v, negT, evt, ect):
    nsteps = B // BLK
    in_specs = [pl.BlockSpec((BLK, D), lambda i: (i, 0))]
    for t in range(T):
        in_specs.append(
            pl.BlockSpec((BLK, D), lambda i, t=t: (t * nsteps + i, 0)))
    in_specs += [
        pl.BlockSpec((BLK, 1), lambda i: (i, 0)),
        pl.BlockSpec((BLK, 1), lambda i: (i, 0)),
        pl.BlockSpec((K, BLK), lambda i: (0, i)),
        pl.BlockSpec((D, BW), lambda i: (0, NBLK)),
        pl.BlockSpec((D, BW), lambda i: (0, NBLK)),
    ]
    return pl.pallas_call(
        _k2_body,
        grid=(nsteps,),
        in_specs=in_specs,
        out_specs=[
            pl.BlockSpec((BLK, D), lambda i: (i, 0)),
            pl.BlockSpec((BW, D), lambda i: (0, 0)),
        ],
        out_shape=[
            jax.ShapeDtypeStruct((B, D), jnp.float32),
            jax.ShapeDtypeStruct((BW, D), jnp.float32),
        ],
    )(vecu, *([vecvf] * T), u[:, None], v[:, None], negT, evt, ect)


@functools.partial(
    pl.kernel,
    out_type=jax.ShapeDtypeStruct((D, N), jnp.float32),
    mesh=_mesh,
    compiler_params=_sc_params,
    scratch_types=[
        pltpu.VMEM((B,), jnp.int32),
        pltpu.VMEM((B + LANES,), jnp.int32),
        pltpu.VMEM((B + LANES,), jnp.int32),
        pltpu.VMEM((B + LANES,), jnp.int32),
        pltpu.VMEM((B + LANES,), jnp.int32),
        pltpu.VMEM((64, BW), jnp.float32),
        pltpu.VMEM((64, BW), jnp.float32),
        pltpu.VMEM((LANES, D), jnp.float32),
        pltpu.SemaphoreType.DMA,
        pltpu.SemaphoreType.DMA,
        pltpu.SemaphoreType.DMA,
    ],
)
def _k3_fused(evt, u, verr, outt, u_all, my_eid, my_row, bl_col, bl_eid,
              bufa, bufb, vring, sem_i, sem_o, sem_v):
    wid = _wid()
    lane = lax.iota(jnp.int32, LANES)

    pltpu.sync_copy(u, u_all)

    def scan_body(i, off):
        uvec = u_all[pl.ds(i * LANES, LANES)]
        msk = ((uvec >> 7) & (NW - 1)) == wid
        plsc.store_compressed(my_eid.at[pl.ds(off, LANES)],
                              lane + i * LANES, mask=msk)
        plsc.store_compressed(my_row.at[pl.ds(off, LANES)], uvec, mask=msk)
        return off + jnp.sum(msk.astype(jnp.int32))

    m = lax.fori_loop(0, B // LANES, scan_body, 0)

    def process(buf, blk):
        def mbody(j, mb):
            mt = my_row[pl.ds(j * LANES, LANES)]
            valid = lane < (m - j * LANES)
            bm = ((mt >> 7) == blk) & valid
            me = my_eid[pl.ds(j * LANES, LANES)]
            plsc.store_compressed(bl_col.at[pl.ds(mb, LANES)],
                                  mt & (BW - 1), mask=bm)
            plsc.store_compressed(bl_eid.at[pl.ds(mb, LANES)], me, mask=bm)
            return mb + jnp.sum(bm.astype(jnp.int32))
        mb = lax.fori_loop(0, (m + LANES - 1) // LANES, mbody, 0)

        @pl.when(mb > 0)
        def _():
            def fire(i, carry):
                e = _ext_at(bl_eid, i, lane)
                pltpu.async_copy(verr.at[pl.ds(e, 1), :],
                                 vring.at[pl.ds(i % LANES, 1), :], sem_v)
                return carry
            lax.fori_loop(0, jnp.minimum(mb, LANES), fire, 0)

            def ebody(i, carry2):
                pltpu.make_async_copy(
                    verr.at[pl.ds(0, 1), :], vring.at[pl.ds(0, 1), :],
                    sem_v).wait()
                c = _ext_at(bl_col, i, lane)
                li = i % LANES
                csp = jnp.full((LANES,), c, jnp.int32)
                for q in range(D // LANES):
                    dvec = lane + q * LANES
                    vals = vring[li, pl.ds(q * LANES, LANES)]
                    plsc.addupdate_scatter(buf, [dvec, csp], vals)
                @pl.when(i + LANES < mb)
                def _():
                    e2 = _ext_at(bl_eid, i + LANES, lane)
                    pltpu.async_copy(verr.at[pl.ds(e2, 1), :],
                                     vring.at[pl.ds(li, 1), :], sem_v)
                return carry2
            lax.fori_loop(0, mb, ebody, 0)

    nfull = NBLK // NW + jnp.where(wid < NBLK % NW, 1, 0)

    @pl.when(nfull > 0)
    def _():
        pltpu.async_copy(evt.at[:, pl.ds(wid * BW, BW)], bufa, sem_i)

        def bbody(bi, carry):
            blk = bi * NW + wid
            nxt = blk + NW

            def go(cur, oth):
                pltpu.make_async_copy(
                    evt.at[:, pl.ds(0, BW)], cur, sem_i).wait()
                @pl.when(bi >= 1)
                def _():
                    pltpu.make_async_copy(
                        cur, outt.at[:, pl.ds(0, BW)], sem_o).wait()
                @pl.when(nxt < NBLK)
                def _():
                    pltpu.async_copy(
                        evt.at[:, pl.ds(nxt * BW, BW)], oth, sem_i)
                process(cur, blk)
                pltpu.async_copy(cur, outt.at[:, pl.ds(blk * BW, BW)], sem_o)

            @pl.when(bi % 2 == 0)
            def _():
                go(bufa, bufb)

            @pl.when(bi % 2 == 1)
            def _():
                go(bufb, bufa)
            return carry

        lax.fori_loop(0, nfull, bbody, 0)

        pltpu.make_async_copy(
            bufa, outt.at[:, pl.ds(0, BW)], sem_o).wait()


def kernel(emb_vertex, emb_context, u, v, neg):
    u = u.astype(jnp.int32)
    v = v.astype(jnp.int32)
    negf = neg.astype(jnp.int32).reshape(-1)
    evt = emb_vertex.T
    ect = emb_context.T
    vecu, vecvf = _k1_sweep(evt, ect, u, v, negf)
    verr, tail = _k2_dense(vecu, vecvf, u, v, neg.astype(jnp.int32).T,
                           evt, ect)
    outt = _k3_fused(evt, u, verr)
    out = outt.T
    return lax.dynamic_update_slice(out, tail[:N - LAST_BASE], (LAST_BASE, 0))


# speedup vs baseline: 1.6865x; 1.3419x over previous
import functools

import jax
import jax.numpy as jnp
from jax import lax
from jax.experimental import pallas as pl
from jax.experimental.pallas import tpu as pltpu
from jax.experimental.pallas import tpu_sc as plsc

ALPHA = 0.025
NC = 2
NS = 16
NW = NC * NS
LANES = 16

N = 1000000
D = 64
B = 16384
K = 5
T = K + 1

EPW = B // NW
NCH = EPW // LANES
ROWS_PW = N // NW

_mesh = plsc.VectorSubcoreMesh(core_axis_name="c", subcore_axis_name="s")
_sc_params = pltpu.CompilerParams(
    use_tc_tiling_on_sc=True, needs_layout_passes=False)

_LANE = None


def _wid():
    return lax.axis_index("s") * NC + lax.axis_index("c")


def _ext(vec, l, lane):
    return jnp.sum(jnp.where(lane == l, vec, jnp.zeros_like(vec)))


@functools.partial(
    pl.kernel,
    out_type=(
        jax.ShapeDtypeStruct((B, D), jnp.float32),
        jax.ShapeDtypeStruct((T, B, D), jnp.float32),
    ),
    mesh=_mesh,
    compiler_params=_sc_params,
    scratch_types=[
        pltpu.VMEM((LANES,), jnp.int32),
        pltpu.VMEM((LANES,), jnp.int32),
        pltpu.VMEM((LANES * K,), jnp.int32),
        pltpu.VMEM((LANES, D), jnp.float32),
        pltpu.VMEM((T, LANES, D), jnp.float32),
        pltpu.SemaphoreType.DMA,
        pltpu.SemaphoreType.DMA,
        pltpu.SemaphoreType.DMA,
    ],
)
def _k1_gather(vertex, context, u, v, negf, vecu_out, vecv_out,
               uv, vv, negv, urows, crows, sem_i, sem_g, sem_w):
    wid = _wid()
    lane = lax.iota(jnp.int32, LANES)

    def chunk_body(c, carry):
        base = wid * EPW + c * LANES
        pltpu.sync_copy(u.at[pl.ds(base, LANES)], uv)
        pltpu.sync_copy(v.at[pl.ds(base, LANES)], vv)
        pltpu.sync_copy(negf.at[pl.ds(base * K, LANES * K)], negv)
        uvec = uv[...]
        vvec = vv[...]
        nblk = [negv[pl.ds(b * LANES, LANES)] for b in range(LANES * K // LANES)]
        fired = []
        for l in range(LANES):
            ul = _ext(uvec, l, lane)
            fired.append(pltpu.async_copy(
                vertex.at[pl.ds(ul, 1), :], urows.at[pl.ds(l, 1), :], sem_g))
            vl = _ext(vvec, l, lane)
            fired.append(pltpu.async_copy(
                context.at[pl.ds(vl, 1), :], crows.at[0, pl.ds(l, 1), :],
                sem_g))
        for t in range(1, T):
            for l in range(LANES):
                fidx = l * K + (t - 1)
                nl = _ext(nblk[fidx // LANES], fidx % LANES, lane)
                fired.append(pltpu.async_copy(
                    context.at[pl.ds(nl, 1), :], crows.at[t, pl.ds(l, 1), :],
                    sem_g))
        for cp in fired:
            cp.wait()
        wr = [pltpu.async_copy(urows, vecu_out.at[pl.ds(base, LANES)], sem_w)]
        for t in range(T):
            wr.append(pltpu.async_copy(
                crows.at[t], vecv_out.at[t, pl.ds(base, LANES)], sem_w))
        for cp in wr:
            cp.wait()
        return carry

    lax.fori_loop(0, NCH, chunk_body, 0)


BLK = 2048


def _k2_body(vecu_ref, vecv_ref, verr_ref):
    vu = vecu_ref[...]
    vv = vecv_ref[...]
    dots = jnp.sum(vv * vu[None, :, :], axis=-1)
    f = 1.0 / (1.0 + jnp.exp(-dots))
    t_idx = lax.broadcasted_iota(jnp.int32, (T, BLK), 0)
    label = jnp.where(t_idx == 0, 1.0, 0.0)
    g = ALPHA * (label - f)
    verr_ref[...] = jnp.sum(g[:, :, None] * vv, axis=0)


def _k2_dense(vecu, vecv):
    return pl.pallas_call(
        _k2_body,
        grid=(B // BLK,),
        in_specs=[
            pl.BlockSpec((BLK, D), lambda i: (i, 0)),
            pl.BlockSpec((T, BLK, D), lambda i: (0, i, 0)),
        ],
        out_specs=pl.BlockSpec((BLK, D), lambda i: (i, 0)),
        out_shape=jax.ShapeDtypeStruct((B, D), jnp.float32),
    )(vecu, vecv)


@functools.partial(
    pl.kernel,
    out_type=(),
    mesh=_mesh,
    compiler_params=_sc_params,
    scratch_types=[
        pltpu.VMEM((B,), jnp.int32),
        pltpu.VMEM((B + LANES,), jnp.int32),
        pltpu.VMEM((B + LANES,), jnp.int32),
        pltpu.VMEM((LANES, D), jnp.float32),
        pltpu.VMEM((LANES, D), jnp.float32),
        pltpu.VMEM((LANES,), jnp.int32),
        pltpu.VMEM((LANES,), jnp.float32),
        pltpu.SemaphoreType.DMA,
        pltpu.SemaphoreType.DMA,
        pltpu.SemaphoreType.DMA,
    ],
)
def _k3_scatter(out_ref, u, verr, u_all, my_eid, my_row, vbuf, obuf,
                rowscr, valscr, sem_v, sem_o, sem_w):
    wid = _wid()
    lo = wid * ROWS_PW
    hi = lo + ROWS_PW
    lane = lax.iota(jnp.int32, LANES)
    perms_f = [(lane + j) % LANES for j in range(LANES)]
    perms_b = [(lane - j) % LANES for j in range(LANES)]

    pltpu.sync_copy(u, u_all)

    def scan_body(i, off):
        uvec = u_all[pl.ds(i * LANES, LANES)]
        msk = (uvec >= lo) & (uvec < hi)
        eids = lane + i * LANES
        plsc.store_compressed(my_eid.at[pl.ds(off, LANES)], eids, mask=msk)
        plsc.store_compressed(my_row.at[pl.ds(off, LANES)], uvec, mask=msk)
        return off + jnp.sum(msk.astype(jnp.int32))

    m = lax.fori_loop(0, B // LANES, scan_body, 0)
    nwaves = (m + LANES - 1) // LANES

    def wave(w, carry):
        r = my_row[pl.ds(w * LANES, LANES)]
        e = my_eid[pl.ds(w * LANES, LANES)]
        validv = lane < (m - w * LANES)
        rmask = jnp.where(validv, r, -1 - lane)

        for l in range(LANES):
            el = _ext(e, l, lane)
            @pl.when(w * LANES + l < m)
            def _(el=el, l=l):
                pltpu.async_copy(
                    verr.at[pl.ds(el, 1), :], vbuf.at[pl.ds(l, 1), :],
                    sem_v)
        rowscr[...] = rmask
        prev_eq = lane < 0
        for j in range(1, LANES):
            rj = plsc.load_gather(rowscr, [perms_b[j]])
            prev_eq = prev_eq | ((rmask == rj) & (lane >= j))
        firstv = jnp.logical_not(prev_eq)
        anyd = jnp.any(prev_eq)
        f01 = jnp.where(firstv, 1, 0)

        for l in range(LANES):
            @pl.when(w * LANES + l < m)
            def _(l=l):
                pltpu.make_async_copy(
                    verr.at[pl.ds(0, 1), :], vbuf.at[pl.ds(l, 1), :],
                    sem_v).wait()

        @pl.when(anyd)
        def _():
            vld01 = jnp.where(validv, 1.0, 0.0)
            fst01 = jnp.where(firstv, 1.0, 0.0)

            def comb_body(d, carry2):
                dsp = jnp.full((LANES,), d, jnp.int32)
                vd = plsc.load_gather(vbuf, [lane, dsp]) * vld01
                valscr[...] = vd
                acc = jnp.zeros((LANES,), jnp.float32)
                for j in range(LANES):
                    rj = plsc.load_gather(rowscr, [perms_f[j]])
                    vj = plsc.load_gather(valscr, [perms_f[j]])
                    acc = acc + jnp.where(rmask == rj, vj, 0.0)
                plsc.store_scatter(vbuf, [lane, dsp], acc * fst01)
                return carry2

            lax.fori_loop(0, D, comb_body, 0)

        rvec = rmask
        for l in range(LANES):
            rl = _ext(rvec, l, lane)
            fl = _ext(f01, l, lane)
            @pl.when((w * LANES + l < m) & (fl == 1))
            def _(rl=rl, l=l):
                pltpu.async_copy(
                    out_ref.at[pl.ds(rl, 1), :], obuf.at[pl.ds(l, 1), :],
                    sem_o)
        for l in range(LANES):
            fl = _ext(f01, l, lane)
            @pl.when((w * LANES + l < m) & (fl == 1))
            def _(l=l):
                pltpu.make_async_copy(
                    out_ref.at[pl.ds(0, 1), :], obuf.at[pl.ds(l, 1), :],
                    sem_o).wait()

        for l in range(LANES):
            for q in range(D // LANES):
                sl = pl.ds(q * LANES, LANES)
                obuf[l, sl] = obuf[l, sl] + vbuf[l, sl]

        for l in range(LANES):
            rl = _ext(rvec, l, lane)
            fl = _ext(f01, l, lane)
            @pl.when((w * LANES + l < m) & (fl == 1))
            def _(rl=rl, l=l):
                pltpu.async_copy(
                    obuf.at[pl.ds(l, 1), :], out_ref.at[pl.ds(rl, 1), :],
                    sem_w)
        for l in range(LANES):
            fl = _ext(f01, l, lane)
            @pl.when((w * LANES + l < m) & (fl == 1))
            def _(l=l):
                pltpu.make_async_copy(
                    obuf.at[pl.ds(l, 1), :], out_ref.at[pl.ds(0, 1), :],
                    sem_w).wait()
        return carry

    lax.fori_loop(0, nwaves, wave, 0)


def kernel(emb_vertex, emb_context, u, v, neg):
    u = u.astype(jnp.int32)
    v = v.astype(jnp.int32)
    negf = neg.astype(jnp.int32).reshape(-1)
    vecu, vecv = _k1_gather(emb_vertex, emb_context, u, v, negf)
    verr = _k2_dense(vecu, vecv)
    out = jax.new_ref(emb_vertex)
    _k3_scatter(out, u, verr)
    return out[...]
